# Initial kernel scaffold; baseline (speedup 1.0000x reference)
import jax
import jax.numpy as jnp
from jax.experimental import pallas as pl


def kernel(embedding, adj_values, edge_index):
    raise NotImplementedError("write your pallas kernel here")


# speedup vs baseline: 2.5333x; 2.5333x over previous
import functools

import jax
import jax.numpy as jnp
from jax import lax
from jax.experimental import pallas as pl
from jax.experimental.pallas import tpu as pltpu
from jax.experimental.pallas import tpu_sc as plsc

N_NODES = 10000
N_PAD = 10240
EMB = 128
N_EDGES = 320000
N_TILES = 16
CHUNK = 128
CHUNKS_PER_TILE = 157
EDGES_PER_TILE = CHUNK * CHUNKS_PER_TILE
EDGES_PAD = EDGES_PER_TILE * N_TILES
ROWS_PER_TILE = N_PAD // N_TILES
ZROWS = 64

_mesh = plsc.VectorSubcoreMesh(core_axis_name="c", subcore_axis_name="s",
                               num_cores=1)


def _scale_rows(gbuf, vals_v):
    def grp_body(g, carry):
        vv = vals_v[pl.ds(g * 16, 16)]
        for j in range(16):
            splat = jnp.full((16,), vv[j], jnp.float32)
            r = g * 16 + j
            for q in range(8):
                sl = pl.ds(q * 16, 16)
                gbuf[r, sl] = gbuf[r, sl] * splat
        return carry
    lax.fori_loop(0, CHUNK // 16, grp_body, 0, unroll=False)


def _layer(tid, src_hbm, rows_hbm, cols_hbm, vals_hbm, acc_sh,
           cidx_v, ridx_v, vals_v, gbuf, sem):
    base0 = tid * EDGES_PER_TILE

    def chunk_body(ci, carry):
        base = base0 + ci * CHUNK
        pltpu.sync_copy(cols_hbm.at[pl.ds(base, CHUNK)], cidx_v)
        pltpu.sync_copy(rows_hbm.at[pl.ds(base, CHUNK)], ridx_v)
        pltpu.sync_copy(vals_hbm.at[pl.ds(base, CHUNK)], vals_v)
        pltpu.async_copy(src_hbm.at[cidx_v], gbuf, sem).wait()
        _scale_rows(gbuf, vals_v)
        pltpu.sync_copy(gbuf, acc_sh.at[ridx_v], add=True)
        return carry

    lax.fori_loop(0, CHUNKS_PER_TILE, chunk_body, 0, unroll=False)


@functools.partial(
    pl.kernel,
    out_type=(
        jax.ShapeDtypeStruct((N_PAD, EMB), jnp.float32),
        jax.ShapeDtypeStruct((N_PAD, EMB), jnp.float32),
    ),
    mesh=_mesh,
    scratch_types=[
        pltpu.VMEM_SHARED((N_PAD, EMB), jnp.float32),
        pltpu.VMEM((CHUNK,), jnp.int32),
        pltpu.VMEM((CHUNK,), jnp.int32),
        pltpu.VMEM((CHUNK,), jnp.float32),
        pltpu.VMEM((CHUNK, EMB), jnp.float32),
        pltpu.VMEM((ZROWS, EMB), jnp.float32),
        pltpu.VMEM((ZROWS, EMB), jnp.float32),
        pltpu.SemaphoreType.DMA,
    ],
)
def _cotrec_kernel(emb_hbm, vals_hbm, rows_hbm, cols_hbm,
                   out_hbm, x1_hbm,
                   acc_sh, cidx_v, ridx_v, vals_v, gbuf, zbuf, fbuf, sem):
    tid = lax.axis_index("s")
    row0 = tid * ROWS_PER_TILE
    zero16 = jnp.zeros((16,), jnp.float32)

    def zrow(r, carry):
        for q in range(8):
            zbuf[r, pl.ds(q * 16, 16)] = zero16
        return carry
    lax.fori_loop(0, ZROWS, zrow, 0, unroll=False)

    def zero_acc_slice():
        for k in range(10):
            pltpu.sync_copy(zbuf, acc_sh.at[pl.ds(row0 + k * ZROWS, ZROWS)])

    zero_acc_slice()
    plsc.subcore_barrier()

    _layer(tid, emb_hbm, rows_hbm, cols_hbm, vals_hbm, acc_sh,
           cidx_v, ridx_v, vals_v, gbuf, sem)
    plsc.subcore_barrier()

    pltpu.sync_copy(acc_sh.at[pl.ds(row0, ROWS_PER_TILE)],
                    x1_hbm.at[pl.ds(row0, ROWS_PER_TILE)])
    zero_acc_slice()
    plsc.subcore_barrier()

    _layer(tid, x1_hbm, rows_hbm, cols_hbm, vals_hbm, acc_sh,
           cidx_v, ridx_v, vals_v, gbuf, sem)
    plsc.subcore_barrier()

    third = jnp.float32(1.0 / 3.0)
    for k in range(10):
        r = row0 + k * ZROWS
        pltpu.sync_copy(emb_hbm.at[pl.ds(r, ZROWS)], gbuf.at[pl.ds(0, ZROWS)])
        pltpu.sync_copy(x1_hbm.at[pl.ds(r, ZROWS)], zbuf)
        pltpu.sync_copy(acc_sh.at[pl.ds(r, ZROWS)], fbuf)

        def frow(rr, carry):
            for q in range(8):
                sl = pl.ds(q * 16, 16)
                gbuf[rr, sl] = (gbuf[rr, sl] + zbuf[rr, sl] + fbuf[rr, sl]) * third
            return carry
        lax.fori_loop(0, ZROWS, frow, 0, unroll=False)

        pltpu.sync_copy(gbuf.at[pl.ds(0, ZROWS)], out_hbm.at[pl.ds(r, ZROWS)])


def kernel(embedding, adj_values, edge_index):
    rows = edge_index[0].astype(jnp.int32)
    cols = edge_index[1].astype(jnp.int32)
    vals = adj_values.astype(jnp.float32)
    pad = EDGES_PAD - N_EDGES
    rows = jnp.pad(rows, (0, pad))
    cols = jnp.pad(cols, (0, pad))
    vals = jnp.pad(vals, (0, pad))
    emb = jnp.pad(embedding.astype(jnp.float32), ((0, N_PAD - N_NODES), (0, 0)))
    out, _ = _cotrec_kernel(emb, vals, rows, cols)
    return out[:N_NODES]


# speedup vs baseline: 2.7826x; 1.0984x over previous
import functools

import jax
import jax.numpy as jnp
from jax import lax
from jax.experimental import pallas as pl
from jax.experimental.pallas import tpu as pltpu
from jax.experimental.pallas import tpu_sc as plsc

N_NODES = 10000
N_PAD = 10240
EMB = 128
N_EDGES = 320000
N_TILES = 16
CHUNK = 128
SUP = 16
NSUP = 10
CHUNKS_PER_TILE = SUP * NSUP
EDGES_PER_TILE = CHUNK * CHUNKS_PER_TILE
EDGES_PAD = EDGES_PER_TILE * N_TILES
N_CHUNKS = EDGES_PAD // CHUNK
ROWS_PER_TILE = N_PAD // N_TILES
ZROWS = 32
FBLK = 64

_mesh = plsc.VectorSubcoreMesh(core_axis_name="c", subcore_axis_name="s",
                               num_cores=1)


def _scale_rows(gbuf, vals2, jj):
    def grp_body(g, carry):
        vv = vals2[jj, pl.ds(g * 16, 16)]
        for k in range(16):
            splat = jnp.full((16,), vv[k], jnp.float32)
            r = g * 16 + k
            for q in range(8):
                sl = pl.ds(q * 16, 16)
                gbuf[r, sl] = gbuf[r, sl] * splat
        return carry
    lax.fori_loop(0, CHUNK // 16, grp_body, 0, unroll=False)


def _layer(tid, src_hbm, rows2_hbm, cols2_hbm, vals2_hbm, acc_sh,
           cidx2, ridx2, vals2, gb0, gb1, sem0, sem1):
    def super_body(j, carry):
        r0 = tid * CHUNKS_PER_TILE + j * SUP
        pltpu.sync_copy(cols2_hbm.at[pl.ds(r0, SUP)], cidx2)
        pltpu.sync_copy(rows2_hbm.at[pl.ds(r0, SUP)], ridx2)
        pltpu.sync_copy(vals2_hbm.at[pl.ds(r0, SUP)], vals2)
        pending = pltpu.async_copy(src_hbm.at[cidx2.at[0]], gb0, sem0)
        for jj in range(SUP):
            cur = gb0 if jj % 2 == 0 else gb1
            if jj < SUP - 1:
                nxt, nsem = (gb1, sem1) if jj % 2 == 0 else (gb0, sem0)
                nxt_pending = pltpu.async_copy(
                    src_hbm.at[cidx2.at[jj + 1]], nxt, nsem)
            pending.wait()
            _scale_rows(cur, vals2, jj)
            pltpu.sync_copy(cur, acc_sh.at[ridx2.at[jj]], add=True)
            if jj < SUP - 1:
                pending = nxt_pending
        return carry

    lax.fori_loop(0, NSUP, super_body, 0, unroll=False)


@functools.partial(
    pl.kernel,
    out_type=(
        jax.ShapeDtypeStruct((N_PAD, EMB), jnp.float32),
        jax.ShapeDtypeStruct((N_PAD, EMB), jnp.float32),
    ),
    mesh=_mesh,
    scratch_types=[
        pltpu.VMEM_SHARED((N_PAD, EMB), jnp.float32),
        pltpu.VMEM((SUP, CHUNK), jnp.int32),
        pltpu.VMEM((SUP, CHUNK), jnp.int32),
        pltpu.VMEM((SUP, CHUNK), jnp.float32),
        pltpu.VMEM((CHUNK, EMB), jnp.float32),
        pltpu.VMEM((CHUNK, EMB), jnp.float32),
        pltpu.VMEM((ZROWS, EMB), jnp.float32),
        pltpu.SemaphoreType.DMA,
        pltpu.SemaphoreType.DMA,
    ],
)
def _cotrec_kernel(emb_hbm, vals2_hbm, rows2_hbm, cols2_hbm,
                   out_hbm, x1_hbm,
                   acc_sh, cidx2, ridx2, vals2, gb0, gb1, zbuf, sem0, sem1):
    tid = lax.axis_index("s")
    row0 = tid * ROWS_PER_TILE
    zero16 = jnp.zeros((16,), jnp.float32)

    def zrow(r, carry):
        for q in range(8):
            zbuf[r, pl.ds(q * 16, 16)] = zero16
        return carry
    lax.fori_loop(0, ZROWS, zrow, 0, unroll=False)

    def zero_acc_slice():
        for k in range(ROWS_PER_TILE // ZROWS):
            pltpu.sync_copy(zbuf, acc_sh.at[pl.ds(row0 + k * ZROWS, ZROWS)])

    zero_acc_slice()
    plsc.subcore_barrier()

    _layer(tid, emb_hbm, rows2_hbm, cols2_hbm, vals2_hbm, acc_sh,
           cidx2, ridx2, vals2, gb0, gb1, sem0, sem1)
    plsc.subcore_barrier()

    pltpu.sync_copy(acc_sh.at[pl.ds(row0, ROWS_PER_TILE)],
                    x1_hbm.at[pl.ds(row0, ROWS_PER_TILE)])
    zero_acc_slice()
    plsc.subcore_barrier()

    _layer(tid, x1_hbm, rows2_hbm, cols2_hbm, vals2_hbm, acc_sh,
           cidx2, ridx2, vals2, gb0, gb1, sem0, sem1)
    plsc.subcore_barrier()

    third = jnp.float32(1.0 / 3.0)
    for k in range(ROWS_PER_TILE // FBLK):
        r = row0 + k * FBLK
        pltpu.sync_copy(emb_hbm.at[pl.ds(r, FBLK)], gb0.at[pl.ds(0, FBLK)])
        pltpu.sync_copy(x1_hbm.at[pl.ds(r, FBLK)], gb0.at[pl.ds(FBLK, FBLK)])
        pltpu.sync_copy(acc_sh.at[pl.ds(r, FBLK)], gb1.at[pl.ds(0, FBLK)])

        def frow(rr, carry):
            for q in range(8):
                sl = pl.ds(q * 16, 16)
                gb1[FBLK + rr, sl] = (gb0[rr, sl] + gb0[FBLK + rr, sl]
                                      + gb1[rr, sl]) * third
            return carry
        lax.fori_loop(0, FBLK, frow, 0, unroll=False)

        pltpu.sync_copy(gb1.at[pl.ds(FBLK, FBLK)], out_hbm.at[pl.ds(r, FBLK)])


def kernel(embedding, adj_values, edge_index):
    rows = edge_index[0].astype(jnp.int32)
    cols = edge_index[1].astype(jnp.int32)
    vals = adj_values.astype(jnp.float32)
    pad = EDGES_PAD - N_EDGES
    rows2 = jnp.pad(rows, (0, pad)).reshape(N_CHUNKS, CHUNK)
    cols2 = jnp.pad(cols, (0, pad)).reshape(N_CHUNKS, CHUNK)
    vals2 = jnp.pad(vals, (0, pad)).reshape(N_CHUNKS, CHUNK)
    emb = jnp.pad(embedding.astype(jnp.float32), ((0, N_PAD - N_NODES), (0, 0)))
    out, _ = _cotrec_kernel(emb, vals2, rows2, cols2)
    return out[:N_NODES]


# speedup vs baseline: 2.7868x; 1.0015x over previous
import functools

import jax
import jax.numpy as jnp
from jax import lax
from jax.experimental import pallas as pl
from jax.experimental.pallas import tpu as pltpu
from jax.experimental.pallas import tpu_sc as plsc

N_NODES = 10000
N_PAD = 10240
EMB = 128
N_EDGES = 320000
N_TILES = 16
CHUNK = 128
SUP = 16
NSUP = 10
CHUNKS_PER_TILE = SUP * NSUP
EDGES_PER_TILE = CHUNK * CHUNKS_PER_TILE
EDGES_PAD = EDGES_PER_TILE * N_TILES
N_CHUNKS = EDGES_PAD // CHUNK
ROWS_PER_TILE = N_PAD // N_TILES
ZROWS = 32
FBLK = 64

_mesh = plsc.VectorSubcoreMesh(core_axis_name="c", subcore_axis_name="s",
                               num_cores=1)


def _scale_rows(gbuf, vals2, jj):
    def grp_body(g, carry):
        vv = vals2[jj, pl.ds(g * 16, 16)]
        for k in range(16):
            splat = jnp.full((16,), vv[k], jnp.float32)
            r = g * 16 + k
            for q in range(8):
                sl = pl.ds(q * 16, 16)
                gbuf[r, sl] = gbuf[r, sl] * splat
        return carry
    lax.fori_loop(0, CHUNK // 16, grp_body, 0, unroll=False)


def _layer(tid, src_hbm, rows2_hbm, cols2_hbm, vals2_hbm, acc_sh,
           cidx2, ridx2, vals2, gb0, gb1, sem0, sem1, ssem0, ssem1):
    gb = (gb0, gb1)
    gsem = (sem0, sem1)
    ssem = (ssem0, ssem1)

    def super_body(j, carry):
        r0 = tid * CHUNKS_PER_TILE + j * SUP
        pltpu.sync_copy(cols2_hbm.at[pl.ds(r0, SUP)], cidx2)
        pltpu.sync_copy(rows2_hbm.at[pl.ds(r0, SUP)], ridx2)
        pltpu.sync_copy(vals2_hbm.at[pl.ds(r0, SUP)], vals2)
        pending = pltpu.async_copy(src_hbm.at[cidx2.at[0]], gb0, sem0)
        scat = [None, None]
        for jj in range(SUP):
            b = jj % 2
            cur = gb[b]
            if jj < SUP - 1:
                b2 = 1 - b
                if scat[b2] is not None:
                    scat[b2].wait()
                    scat[b2] = None
                nxt_pending = pltpu.async_copy(
                    src_hbm.at[cidx2.at[jj + 1]], gb[b2], gsem[b2])
            pending.wait()
            _scale_rows(cur, vals2, jj)
            scat[b] = pltpu.async_copy(cur, acc_sh.at[ridx2.at[jj]], ssem[b],
                                       add=True)
            if jj < SUP - 1:
                pending = nxt_pending
        for b in range(2):
            if scat[b] is not None:
                scat[b].wait()
        return carry

    lax.fori_loop(0, NSUP, super_body, 0, unroll=False)


@functools.partial(
    pl.kernel,
    out_type=(
        jax.ShapeDtypeStruct((N_PAD, EMB), jnp.float32),
        jax.ShapeDtypeStruct((N_PAD, EMB), jnp.float32),
    ),
    mesh=_mesh,
    scratch_types=[
        pltpu.VMEM_SHARED((N_PAD, EMB), jnp.float32),
        pltpu.VMEM((SUP, CHUNK), jnp.int32),
        pltpu.VMEM((SUP, CHUNK), jnp.int32),
        pltpu.VMEM((SUP, CHUNK), jnp.float32),
        pltpu.VMEM((CHUNK, EMB), jnp.float32),
        pltpu.VMEM((CHUNK, EMB), jnp.float32),
        pltpu.VMEM((ZROWS, EMB), jnp.float32),
        pltpu.SemaphoreType.DMA,
        pltpu.SemaphoreType.DMA,
        pltpu.SemaphoreType.DMA,
        pltpu.SemaphoreType.DMA,
    ],
)
def _cotrec_kernel(emb_hbm, vals2_hbm, rows2_hbm, cols2_hbm,
                   out_hbm, x1_hbm,
                   acc_sh, cidx2, ridx2, vals2, gb0, gb1, zbuf, sem0, sem1, ssem0, ssem1):
    tid = lax.axis_index("s")
    row0 = tid * ROWS_PER_TILE
    zero16 = jnp.zeros((16,), jnp.float32)

    def zrow(r, carry):
        for q in range(8):
            zbuf[r, pl.ds(q * 16, 16)] = zero16
        return carry
    lax.fori_loop(0, ZROWS, zrow, 0, unroll=False)

    def zero_acc_slice():
        for k in range(ROWS_PER_TILE // ZROWS):
            pltpu.sync_copy(zbuf, acc_sh.at[pl.ds(row0 + k * ZROWS, ZROWS)])

    zero_acc_slice()
    plsc.subcore_barrier()

    _layer(tid, emb_hbm, rows2_hbm, cols2_hbm, vals2_hbm, acc_sh,
           cidx2, ridx2, vals2, gb0, gb1, sem0, sem1, ssem0, ssem1)
    plsc.subcore_barrier()

    pltpu.sync_copy(acc_sh.at[pl.ds(row0, ROWS_PER_TILE)],
                    x1_hbm.at[pl.ds(row0, ROWS_PER_TILE)])
    zero_acc_slice()
    plsc.subcore_barrier()

    _layer(tid, x1_hbm, rows2_hbm, cols2_hbm, vals2_hbm, acc_sh,
           cidx2, ridx2, vals2, gb0, gb1, sem0, sem1, ssem0, ssem1)
    plsc.subcore_barrier()

    third = jnp.float32(1.0 / 3.0)
    for k in range(ROWS_PER_TILE // FBLK):
        r = row0 + k * FBLK
        pltpu.sync_copy(emb_hbm.at[pl.ds(r, FBLK)], gb0.at[pl.ds(0, FBLK)])
        pltpu.sync_copy(x1_hbm.at[pl.ds(r, FBLK)], gb0.at[pl.ds(FBLK, FBLK)])
        pltpu.sync_copy(acc_sh.at[pl.ds(r, FBLK)], gb1.at[pl.ds(0, FBLK)])

        def frow(rr, carry):
            for q in range(8):
                sl = pl.ds(q * 16, 16)
                gb1[FBLK + rr, sl] = (gb0[rr, sl] + gb0[FBLK + rr, sl]
                                      + gb1[rr, sl]) * third
            return carry
        lax.fori_loop(0, FBLK, frow, 0, unroll=False)

        pltpu.sync_copy(gb1.at[pl.ds(FBLK, FBLK)], out_hbm.at[pl.ds(r, FBLK)])


def kernel(embedding, adj_values, edge_index):
    rows = edge_index[0].astype(jnp.int32)
    cols = edge_index[1].astype(jnp.int32)
    vals = adj_values.astype(jnp.float32)
    pad = EDGES_PAD - N_EDGES
    rows2 = jnp.pad(rows, (0, pad)).reshape(N_CHUNKS, CHUNK)
    cols2 = jnp.pad(cols, (0, pad)).reshape(N_CHUNKS, CHUNK)
    vals2 = jnp.pad(vals, (0, pad)).reshape(N_CHUNKS, CHUNK)
    emb = jnp.pad(embedding.astype(jnp.float32), ((0, N_PAD - N_NODES), (0, 0)))
    out, _ = _cotrec_kernel(emb, vals2, rows2, cols2)
    return out[:N_NODES]


# speedup vs baseline: 2.8741x; 1.0313x over previous
import functools

import jax
import jax.numpy as jnp
from jax import lax
from jax.experimental import pallas as pl
from jax.experimental.pallas import tpu as pltpu
from jax.experimental.pallas import tpu_sc as plsc

N_NODES = 10000
N_PAD = 10240
EMB = 128
N_EDGES = 320000
N_CORES = 2
N_TILES = 16
N_WORKERS = N_CORES * N_TILES
CHUNK = 128
SUP = 16
NSUP = 5
CHUNKS_PER_TILE = SUP * NSUP
CHUNKS_PER_CORE = CHUNKS_PER_TILE * N_TILES
N_CHUNKS = CHUNKS_PER_CORE * N_CORES
EDGES_PAD = N_CHUNKS * CHUNK
ROWS_PER_TILE = N_PAD // N_TILES
ROWS_PER_WORKER = N_PAD // N_WORKERS
ZROWS = 32
FBLK = 64

_mesh = plsc.VectorSubcoreMesh(core_axis_name="c", subcore_axis_name="s",
                               num_cores=N_CORES)


def _scale_rows(gbuf, vals2, jj):
    def grp_body(g, carry):
        vv = vals2[jj, pl.ds(g * 16, 16)]
        for k in range(16):
            splat = jnp.full((16,), vv[k], jnp.float32)
            r = g * 16 + k
            for q in range(8):
                sl = pl.ds(q * 16, 16)
                gbuf[r, sl] = gbuf[r, sl] * splat
        return carry
    lax.fori_loop(0, CHUNK // 16, grp_body, 0, unroll=False)


def _layer(cid, tid, src_hbm, rows2_hbm, cols2_hbm, vals2_hbm, acc_sh,
           cidx2, ridx2, vals2, gb0, gb1, sem0, sem1, ssem0, ssem1):
    gb = (gb0, gb1)
    gsem = (sem0, sem1)
    ssem = (ssem0, ssem1)

    def super_body(j, carry):
        r0 = cid * CHUNKS_PER_CORE + tid * CHUNKS_PER_TILE + j * SUP
        pltpu.sync_copy(cols2_hbm.at[pl.ds(r0, SUP)], cidx2)
        pltpu.sync_copy(rows2_hbm.at[pl.ds(r0, SUP)], ridx2)
        pltpu.sync_copy(vals2_hbm.at[pl.ds(r0, SUP)], vals2)
        pending = pltpu.async_copy(src_hbm.at[cidx2.at[0]], gb0, sem0)
        scat = [None, None]
        for jj in range(SUP):
            b = jj % 2
            cur = gb[b]
            if jj < SUP - 1:
                b2 = 1 - b
                if scat[b2] is not None:
                    scat[b2].wait()
                    scat[b2] = None
                nxt_pending = pltpu.async_copy(
                    src_hbm.at[cidx2.at[jj + 1]], gb[b2], gsem[b2])
            pending.wait()
            _scale_rows(cur, vals2, jj)
            scat[b] = pltpu.async_copy(cur, acc_sh.at[ridx2.at[jj]], ssem[b],
                                       add=True)
            if jj < SUP - 1:
                pending = nxt_pending
        for b in range(2):
            if scat[b] is not None:
                scat[b].wait()
        return carry

    lax.fori_loop(0, NSUP, super_body, 0, unroll=False)


@functools.partial(
    pl.kernel,
    out_type=jax.ShapeDtypeStruct((N_CORES, N_PAD, EMB), jnp.float32),
    mesh=_mesh,
    scratch_types=[
        pltpu.VMEM_SHARED((N_PAD, EMB), jnp.float32),
        pltpu.VMEM((SUP, CHUNK), jnp.int32),
        pltpu.VMEM((SUP, CHUNK), jnp.int32),
        pltpu.VMEM((SUP, CHUNK), jnp.float32),
        pltpu.VMEM((CHUNK, EMB), jnp.float32),
        pltpu.VMEM((CHUNK, EMB), jnp.float32),
        pltpu.VMEM((ZROWS, EMB), jnp.float32),
        pltpu.SemaphoreType.DMA,
        pltpu.SemaphoreType.DMA,
        pltpu.SemaphoreType.DMA,
        pltpu.SemaphoreType.DMA,
    ],
)
def _spmm_partial(src_hbm, vals2_hbm, rows2_hbm, cols2_hbm, part_hbm,
                  acc_sh, cidx2, ridx2, vals2, gb0, gb1, zbuf,
                  sem0, sem1, ssem0, ssem1):
    cid = lax.axis_index("c")
    tid = lax.axis_index("s")
    row0 = tid * ROWS_PER_TILE
    zero16 = jnp.zeros((16,), jnp.float32)

    def zrow(r, carry):
        for q in range(8):
            zbuf[r, pl.ds(q * 16, 16)] = zero16
        return carry
    lax.fori_loop(0, ZROWS, zrow, 0, unroll=False)

    for k in range(ROWS_PER_TILE // ZROWS):
        pltpu.sync_copy(zbuf, acc_sh.at[pl.ds(row0 + k * ZROWS, ZROWS)])
    plsc.subcore_barrier()

    _layer(cid, tid, src_hbm, rows2_hbm, cols2_hbm, vals2_hbm, acc_sh,
           cidx2, ridx2, vals2, gb0, gb1, sem0, sem1, ssem0, ssem1)
    plsc.subcore_barrier()

    pltpu.sync_copy(acc_sh.at[pl.ds(row0, ROWS_PER_TILE)],
                    part_hbm.at[cid].at[pl.ds(row0, ROWS_PER_TILE)])


@functools.partial(
    pl.kernel,
    out_type=jax.ShapeDtypeStruct((N_PAD, EMB), jnp.float32),
    mesh=_mesh,
    scratch_types=[
        pltpu.VMEM((FBLK, EMB), jnp.float32),
        pltpu.VMEM((FBLK, EMB), jnp.float32),
    ],
)
def _combine2(part_hbm, x1_hbm, bufa, bufb):
    wid = lax.axis_index("c") * N_TILES + lax.axis_index("s")
    row0 = wid * ROWS_PER_WORKER
    for k in range(ROWS_PER_WORKER // FBLK):
        r = row0 + k * FBLK
        pltpu.sync_copy(part_hbm.at[0].at[pl.ds(r, FBLK)], bufa)
        pltpu.sync_copy(part_hbm.at[1].at[pl.ds(r, FBLK)], bufb)

        def frow(rr, carry):
            for q in range(8):
                sl = pl.ds(q * 16, 16)
                bufa[rr, sl] = bufa[rr, sl] + bufb[rr, sl]
            return carry
        lax.fori_loop(0, FBLK, frow, 0, unroll=False)

        pltpu.sync_copy(bufa, x1_hbm.at[pl.ds(r, FBLK)])


@functools.partial(
    pl.kernel,
    out_type=jax.ShapeDtypeStruct((N_PAD, EMB), jnp.float32),
    mesh=_mesh,
    scratch_types=[
        pltpu.VMEM((FBLK, EMB), jnp.float32),
        pltpu.VMEM((FBLK, EMB), jnp.float32),
    ],
)
def _final4(emb_hbm, x1_hbm, part_hbm, out_hbm, bufa, bufb):
    wid = lax.axis_index("c") * N_TILES + lax.axis_index("s")
    row0 = wid * ROWS_PER_WORKER
    third = jnp.float32(1.0 / 3.0)
    for k in range(ROWS_PER_WORKER // FBLK):
        r = row0 + k * FBLK
        pltpu.sync_copy(emb_hbm.at[pl.ds(r, FBLK)], bufa)
        pltpu.sync_copy(x1_hbm.at[pl.ds(r, FBLK)], bufb)

        def add_rows(rr, carry):
            for q in range(8):
                sl = pl.ds(q * 16, 16)
                bufa[rr, sl] = bufa[rr, sl] + bufb[rr, sl]
            return carry

        lax.fori_loop(0, FBLK, add_rows, 0, unroll=False)
        pltpu.sync_copy(part_hbm.at[0].at[pl.ds(r, FBLK)], bufb)
        lax.fori_loop(0, FBLK, add_rows, 0, unroll=False)
        pltpu.sync_copy(part_hbm.at[1].at[pl.ds(r, FBLK)], bufb)

        def fin_rows(rr, carry):
            for q in range(8):
                sl = pl.ds(q * 16, 16)
                bufa[rr, sl] = (bufa[rr, sl] + bufb[rr, sl]) * third
            return carry
        lax.fori_loop(0, FBLK, fin_rows, 0, unroll=False)

        pltpu.sync_copy(bufa, out_hbm.at[pl.ds(r, FBLK)])


def kernel(embedding, adj_values, edge_index):
    rows = edge_index[0].astype(jnp.int32)
    cols = edge_index[1].astype(jnp.int32)
    vals = adj_values.astype(jnp.float32)
    pad = EDGES_PAD - N_EDGES
    rows2 = jnp.pad(rows, (0, pad)).reshape(N_CHUNKS, CHUNK)
    cols2 = jnp.pad(cols, (0, pad)).reshape(N_CHUNKS, CHUNK)
    vals2 = jnp.pad(vals, (0, pad)).reshape(N_CHUNKS, CHUNK)
    emb = jnp.pad(embedding.astype(jnp.float32), ((0, N_PAD - N_NODES), (0, 0)))
    parts1 = _spmm_partial(emb, vals2, rows2, cols2)
    x1 = _combine2(parts1)
    parts2 = _spmm_partial(x1, vals2, rows2, cols2)
    out = _final4(emb, x1, parts2)
    return out[:N_NODES]


# speedup vs baseline: 4.9965x; 1.7385x over previous
import functools

import jax
import jax.numpy as jnp
from jax import lax
from jax.experimental import pallas as pl
from jax.experimental.pallas import tpu as pltpu
from jax.experimental.pallas import tpu_sc as plsc

N_NODES = 10000
N_PAD = 10240
EMB = 128
HALF = EMB // 2
N_HALVES = 2
N_EDGES = 320000
N_CORES = 2
N_TILES = 16
N_WORKERS = N_CORES * N_TILES
CHUNK = 128
SUP = 16
NSUP = 5
CHUNKS_PER_TILE = SUP * NSUP
CHUNKS_PER_CORE = CHUNKS_PER_TILE * N_TILES
N_CHUNKS = CHUNKS_PER_CORE * N_CORES
EDGES_PAD = N_CHUNKS * CHUNK
ROWS_PER_TILE = N_PAD // N_TILES
ROWS_PER_WORKER = N_PAD // N_WORKERS
ZROWS = 64
FBLK = 64

_mesh = plsc.VectorSubcoreMesh(core_axis_name="c", subcore_axis_name="s",
                               num_cores=N_CORES)


def _scale_rows(gbuf, vals2, jj):
    def grp_body(g, carry):
        vv = vals2[jj, pl.ds(g * 16, 16)]
        for k in range(16):
            splat = jnp.full((16,), vv[k], jnp.float32)
            r = g * 16 + k
            for q in range(HALF // 16):
                sl = pl.ds(q * 16, 16)
                gbuf[r, sl] = gbuf[r, sl] * splat
        return carry
    lax.fori_loop(0, CHUNK // 16, grp_body, 0, unroll=False)


def _layer(cid, tid, xs_sh, rows2_hbm, cols2_hbm, vals2_hbm, acc_sh,
           cidx2, ridx2, vals2, gb0, gb1, sem0, sem1, ssem0, ssem1):
    gb = (gb0, gb1)
    gsem = (sem0, sem1)
    ssem = (ssem0, ssem1)

    def super_body(j, carry):
        r0 = cid * CHUNKS_PER_CORE + tid * CHUNKS_PER_TILE + j * SUP
        pltpu.sync_copy(cols2_hbm.at[pl.ds(r0, SUP)], cidx2)
        pltpu.sync_copy(rows2_hbm.at[pl.ds(r0, SUP)], ridx2)
        pltpu.sync_copy(vals2_hbm.at[pl.ds(r0, SUP)], vals2)
        pending = pltpu.async_copy(xs_sh.at[cidx2.at[0]], gb0, sem0)
        scat = [None, None]
        for jj in range(SUP):
            b = jj % 2
            cur = gb[b]
            if jj < SUP - 1:
                b2 = 1 - b
                if scat[b2] is not None:
                    scat[b2].wait()
                    scat[b2] = None
                nxt_pending = pltpu.async_copy(
                    xs_sh.at[cidx2.at[jj + 1]], gb[b2], gsem[b2])
            pending.wait()
            _scale_rows(cur, vals2, jj)
            scat[b] = pltpu.async_copy(cur, acc_sh.at[ridx2.at[jj]], ssem[b],
                                       add=True)
            if jj < SUP - 1:
                pending = nxt_pending
        for b in range(2):
            if scat[b] is not None:
                scat[b].wait()
        return carry

    lax.fori_loop(0, NSUP, super_body, 0, unroll=False)


@functools.partial(
    pl.kernel,
    out_type=jax.ShapeDtypeStruct((N_CORES, N_HALVES, N_PAD, HALF),
                                  jnp.float32),
    mesh=_mesh,
    scratch_types=[
        pltpu.VMEM_SHARED((N_PAD, HALF), jnp.float32),
        pltpu.VMEM_SHARED((N_PAD, HALF), jnp.float32),
        pltpu.VMEM((SUP, CHUNK), jnp.int32),
        pltpu.VMEM((SUP, CHUNK), jnp.int32),
        pltpu.VMEM((SUP, CHUNK), jnp.float32),
        pltpu.VMEM((CHUNK, HALF), jnp.float32),
        pltpu.VMEM((CHUNK, HALF), jnp.float32),
        pltpu.VMEM((ZROWS, HALF), jnp.float32),
        pltpu.SemaphoreType.DMA,
        pltpu.SemaphoreType.DMA,
        pltpu.SemaphoreType.DMA,
        pltpu.SemaphoreType.DMA,
    ],
)
def _spmm_partial(src_hbm, vals2_hbm, rows2_hbm, cols2_hbm, part_hbm,
                  xs_sh, acc_sh, cidx2, ridx2, vals2, gb0, gb1, zbuf,
                  sem0, sem1, ssem0, ssem1):
    cid = lax.axis_index("c")
    tid = lax.axis_index("s")
    row0 = tid * ROWS_PER_TILE
    zero16 = jnp.zeros((16,), jnp.float32)

    def zrow(r, carry):
        for q in range(HALF // 16):
            zbuf[r, pl.ds(q * 16, 16)] = zero16
        return carry
    lax.fori_loop(0, ZROWS, zrow, 0, unroll=False)

    for h in range(N_HALVES):
        pltpu.sync_copy(src_hbm.at[h].at[pl.ds(row0, ROWS_PER_TILE)],
                        xs_sh.at[pl.ds(row0, ROWS_PER_TILE)])
        for k in range(ROWS_PER_TILE // ZROWS):
            pltpu.sync_copy(zbuf, acc_sh.at[pl.ds(row0 + k * ZROWS, ZROWS)])
        plsc.subcore_barrier()

        _layer(cid, tid, xs_sh, rows2_hbm, cols2_hbm, vals2_hbm, acc_sh,
               cidx2, ridx2, vals2, gb0, gb1, sem0, sem1, ssem0, ssem1)
        plsc.subcore_barrier()

        pltpu.sync_copy(acc_sh.at[pl.ds(row0, ROWS_PER_TILE)],
                        part_hbm.at[cid].at[h].at[pl.ds(row0, ROWS_PER_TILE)])
        if h + 1 < N_HALVES:
            plsc.subcore_barrier()


@functools.partial(
    pl.kernel,
    out_type=jax.ShapeDtypeStruct((N_HALVES, N_PAD, HALF), jnp.float32),
    mesh=_mesh,
    scratch_types=[
        pltpu.VMEM((FBLK, HALF), jnp.float32),
        pltpu.VMEM((FBLK, HALF), jnp.float32),
    ],
)
def _combine2(part_hbm, x1_hbm, bufa, bufb):
    wid = lax.axis_index("c") * N_TILES + lax.axis_index("s")
    row0 = wid * ROWS_PER_WORKER
    for h in range(N_HALVES):
        for k in range(ROWS_PER_WORKER // FBLK):
            r = row0 + k * FBLK
            pltpu.sync_copy(part_hbm.at[0].at[h].at[pl.ds(r, FBLK)], bufa)
            pltpu.sync_copy(part_hbm.at[1].at[h].at[pl.ds(r, FBLK)], bufb)

            def frow(rr, carry):
                for q in range(HALF // 16):
                    sl = pl.ds(q * 16, 16)
                    bufa[rr, sl] = bufa[rr, sl] + bufb[rr, sl]
                return carry
            lax.fori_loop(0, FBLK, frow, 0, unroll=False)

            pltpu.sync_copy(bufa, x1_hbm.at[h].at[pl.ds(r, FBLK)])


@functools.partial(
    pl.kernel,
    out_type=jax.ShapeDtypeStruct((N_HALVES, N_PAD, HALF), jnp.float32),
    mesh=_mesh,
    scratch_types=[
        pltpu.VMEM((FBLK, HALF), jnp.float32),
        pltpu.VMEM((FBLK, HALF), jnp.float32),
    ],
)
def _final4(emb_hbm, x1_hbm, part_hbm, out_hbm, bufa, bufb):
    wid = lax.axis_index("c") * N_TILES + lax.axis_index("s")
    row0 = wid * ROWS_PER_WORKER
    third = jnp.float32(1.0 / 3.0)
    for h in range(N_HALVES):
        for k in range(ROWS_PER_WORKER // FBLK):
            r = row0 + k * FBLK
            pltpu.sync_copy(emb_hbm.at[h].at[pl.ds(r, FBLK)], bufa)
            pltpu.sync_copy(x1_hbm.at[h].at[pl.ds(r, FBLK)], bufb)

            def add_rows(rr, carry):
                for q in range(HALF // 16):
                    sl = pl.ds(q * 16, 16)
                    bufa[rr, sl] = bufa[rr, sl] + bufb[rr, sl]
                return carry

            lax.fori_loop(0, FBLK, add_rows, 0, unroll=False)
            pltpu.sync_copy(part_hbm.at[0].at[h].at[pl.ds(r, FBLK)], bufb)
            lax.fori_loop(0, FBLK, add_rows, 0, unroll=False)
            pltpu.sync_copy(part_hbm.at[1].at[h].at[pl.ds(r, FBLK)], bufb)

            def fin_rows(rr, carry):
                for q in range(HALF // 16):
                    sl = pl.ds(q * 16, 16)
                    bufa[rr, sl] = (bufa[rr, sl] + bufb[rr, sl]) * third
                return carry
            lax.fori_loop(0, FBLK, fin_rows, 0, unroll=False)

            pltpu.sync_copy(bufa, out_hbm.at[h].at[pl.ds(r, FBLK)])


def kernel(embedding, adj_values, edge_index):
    rows = edge_index[0].astype(jnp.int32)
    cols = edge_index[1].astype(jnp.int32)
    vals = adj_values.astype(jnp.float32)
    pad = EDGES_PAD - N_EDGES
    rows2 = jnp.pad(rows, (0, pad)).reshape(N_CHUNKS, CHUNK)
    cols2 = jnp.pad(cols, (0, pad)).reshape(N_CHUNKS, CHUNK)
    vals2 = jnp.pad(vals, (0, pad)).reshape(N_CHUNKS, CHUNK)
    emb = jnp.pad(embedding.astype(jnp.float32), ((0, N_PAD - N_NODES), (0, 0)))
    emb_s = emb.reshape(N_PAD, N_HALVES, HALF).transpose(1, 0, 2)
    parts1 = _spmm_partial(emb_s, vals2, rows2, cols2)
    x1_s = _combine2(parts1)
    parts2 = _spmm_partial(x1_s, vals2, rows2, cols2)
    out_s = _final4(emb_s, x1_s, parts2)
    out = out_s.transpose(1, 0, 2).reshape(N_PAD, EMB)
    return out[:N_NODES]
